# baseline (speedup 1.0000x reference)
import functools

import jax
import jax.numpy as jnp
from jax import lax
from jax.experimental import pallas as pl
from jax.experimental.pallas import tpu as pltpu
from jax.experimental.pallas import tpu_sc as plsc

N = 10000
E = 160000
D_IN = 256
D_HID = 512

N_PAD = 10240
N_PAD_ROWS = N_PAD - N
LANES = 128
N_SUBCORES = 16
N_CORES = 2
BPT = -(-E // (N_SUBCORES * LANES * 8)) * 8
E_BATCHES = BPT * N_SUBCORES
E_PAD = E_BATCHES * LANES
RPT = N_PAD // N_SUBCORES
IBLK = 16
N_IBLK = BPT // IBLK
XSUB = RPT // LANES

_MESH = plsc.VectorSubcoreMesh(
    core_axis_name="c", subcore_axis_name="s",
    num_cores=N_CORES, num_subcores=N_SUBCORES)


def _rsqrt16(x):
  i = lax.bitcast_convert_type(x, jnp.int32)
  i = 0x5F3759DF - lax.shift_right_arithmetic(i, 1)
  y = lax.bitcast_convert_type(i, jnp.float32)
  for _ in range(3):
    y = y * (1.5 - 0.5 * x * y * y)
  return y


def _agg_chunk(tab, srcp, dstp, out_slice, acc_sp, idxp, rows, gsems, ssems,
               isem, erow0, r0):
  pltpu.sync_copy(tab.at[pl.ds(r0, RPT), :], acc_sp.at[pl.ds(r0, RPT), :])
  plsc.subcore_barrier()

  _edge_pipeline(tab, acc_sp, srcp, dstp, idxp, rows, gsems, ssems, isem,
                 erow0)

  plsc.subcore_barrier()
  pltpu.sync_copy(acc_sp.at[pl.ds(r0, RPT), :], out_slice)
  plsc.subcore_barrier()


def _edge_pipeline(tab, accum, srcp, dstp, idxp, bufs, gsems, ssems, isem,
                   erow0):
  (sidx0, didx0), (sidx1, didx1) = idxp
  sidxs = (sidx0, sidx1)
  didxs = (didx0, didx1)
  pltpu.sync_copy(srcp.at[pl.ds(erow0, IBLK), :], sidx0)
  pltpu.sync_copy(dstp.at[pl.ds(erow0, IBLK), :], didx0)

  gd = [None, None]
  sd = [None, None]
  id_ = [None, None]
  gd[0] = pltpu.async_copy(tab.at[sidx0.at[0]], bufs[0], gsems[0])
  for blk in range(N_IBLK):
    cur = blk % 2
    if blk >= 1:
      sd[(blk * IBLK - 1) % 2].wait()
    if blk + 1 < N_IBLK:
      nxt = (blk + 1) % 2
      off = erow0 + (blk + 1) * IBLK
      id_[0] = pltpu.async_copy(srcp.at[pl.ds(off, IBLK), :], sidxs[nxt], isem)
      id_[1] = pltpu.async_copy(dstp.at[pl.ds(off, IBLK), :], didxs[nxt], isem)
    for b in range(IBLK):
      k = blk * IBLK + b
      if k >= 1 and b >= 1:
        sd[(k - 1) % 2].wait()
      if k + 1 < BPT:
        if b + 1 < IBLK:
          nsidx, nb = sidxs[cur], b + 1
        else:
          id_[0].wait()
          id_[1].wait()
          nsidx, nb = sidxs[(blk + 1) % 2], 0
        gd[(k + 1) % 2] = pltpu.async_copy(
            tab.at[nsidx.at[nb]], bufs[(k + 1) % 2], gsems[(k + 1) % 2])
      gd[k % 2].wait()
      sd[k % 2] = pltpu.async_copy(
          bufs[k % 2], accum.at[didxs[cur].at[b]], ssems[k % 2], add=True)
  sd[(BPT - 1) % 2].wait()


def _ka_body(xpad, srcp, dstp, dis_hbm, s_hbm, y1_hbm, g1_hbm,
             acc_sp, deg_sp, t_sp, dis_sp,
             sidx, didx, sidxb, didxb, rows0, rows1, ones_v, vals_v, vals_b,
             buf_a, buf_b, gsem0, gsem1, ssem0, ssem1, isem):
  c = lax.axis_index("c")
  sid = lax.axis_index("s")
  erow0 = sid * BPT
  r0 = sid * RPT
  rows = (rows0, rows1)
  idxp = ((sidx, didx), (sidxb, didxb))
  gsems = (gsem0, gsem1)
  ssems = (ssem0, ssem1)

  @pl.loop(0, RPT // 16)
  def _zero(i):
    buf_a[pl.ds(i * 16, 16)] = jnp.zeros((16,), jnp.float32)

  pltpu.sync_copy(buf_a, deg_sp.at[pl.ds(r0, RPT)])
  pltpu.sync_copy(buf_a, t_sp.at[pl.ds(r0, RPT)])
  for i in range(LANES // 16):
    ones_v[pl.ds(i * 16, 16)] = jnp.ones((16,), jnp.float32)
  plsc.subcore_barrier()

  didxs = (didx, didxb)
  pltpu.sync_copy(dstp.at[pl.ds(erow0, IBLK), :], didxs[0])
  for blk in range(N_IBLK):
    cur = blk % 2
    idxd = None
    if blk + 1 < N_IBLK:
      idxd = pltpu.async_copy(
          dstp.at[pl.ds(erow0 + (blk + 1) * IBLK, IBLK), :],
          didxs[(blk + 1) % 2], isem)
    descs = [pltpu.async_copy(ones_v, deg_sp.at[didxs[cur].at[b]], ssem0,
                              add=True)
             for b in range(IBLK)]
    for d in descs:
      d.wait()
    if idxd is not None:
      idxd.wait()
  plsc.subcore_barrier()

  pltpu.sync_copy(deg_sp.at[pl.ds(r0, RPT)], buf_a)

  @pl.loop(0, RPT // 16)
  def _dis(i):
    d = buf_a[pl.ds(i * 16, 16)] + 1.0
    buf_b[pl.ds(i * 16, 16)] = _rsqrt16(d)

  pltpu.sync_copy(buf_b, dis_sp.at[pl.ds(r0, RPT)])

  @pl.when(c == 0)
  def _():
    pltpu.sync_copy(buf_b, dis_hbm.at[pl.ds(r0, RPT)])

  plsc.subcore_barrier()

  _edge_pipeline(dis_sp, t_sp, srcp, dstp, idxp, (vals_v, vals_b),
                 gsems, ssems, isem, erow0)
  plsc.subcore_barrier()

  pltpu.sync_copy(t_sp.at[pl.ds(r0, RPT)], buf_a)

  @pl.loop(0, RPT // 16)
  def _s(i):
    d = buf_b[pl.ds(i * 16, 16)]
    buf_a[pl.ds(i * 16, 16)] = d * (buf_a[pl.ds(i * 16, 16)] + d)

  @pl.when(c == 0)
  def _():
    pltpu.sync_copy(buf_a, s_hbm.at[pl.ds(r0, RPT)])

  for sub in range(XSUB):
    rbase = r0 + sub * LANES
    pltpu.sync_copy(
        xpad.at[pl.ds(rbase, LANES), pl.ds(c * LANES, LANES)], rows0)

    @pl.loop(0, LANES // 16)
    def _scale(rg):
      d16 = buf_b[pl.ds(sub * LANES + rg * 16, 16)]
      for l in range(16):
        d = d16[l]
        rr = rg * 16 + l
        for j in range(LANES // 16):
          rows0[rr, pl.ds(j * 16, 16)] = rows0[rr, pl.ds(j * 16, 16)] * d

    pltpu.sync_copy(rows0, y1_hbm.at[c, pl.ds(rbase, LANES), :])
  plsc.subcore_barrier()

  _agg_chunk(y1_hbm.at[c], srcp, dstp, g1_hbm.at[c, pl.ds(r0, RPT), :],
             acc_sp, idxp, rows, gsems, ssems, isem, erow0, r0)


_ka = pl.kernel(
    _ka_body,
    out_type=[jax.ShapeDtypeStruct((N_PAD,), jnp.float32),
              jax.ShapeDtypeStruct((N_PAD,), jnp.float32),
              jax.ShapeDtypeStruct((D_IN // LANES, N_PAD, LANES), jnp.float32),
              jax.ShapeDtypeStruct((D_IN // LANES, N_PAD, LANES), jnp.float32)],
    mesh=_MESH,
    scratch_types=[
        pltpu.VMEM_SHARED((N_PAD, LANES), jnp.float32),
        pltpu.VMEM_SHARED((N_PAD,), jnp.float32),
        pltpu.VMEM_SHARED((N_PAD,), jnp.float32),
        pltpu.VMEM_SHARED((N_PAD,), jnp.float32),
        pltpu.VMEM((IBLK, LANES), jnp.int32),
        pltpu.VMEM((IBLK, LANES), jnp.int32),
        pltpu.VMEM((IBLK, LANES), jnp.int32),
        pltpu.VMEM((IBLK, LANES), jnp.int32),
        pltpu.VMEM((LANES, LANES), jnp.float32),
        pltpu.VMEM((LANES, LANES), jnp.float32),
        pltpu.VMEM((LANES,), jnp.float32),
        pltpu.VMEM((LANES,), jnp.float32),
        pltpu.VMEM((LANES,), jnp.float32),
        pltpu.VMEM((RPT,), jnp.float32),
        pltpu.VMEM((RPT,), jnp.float32),
        pltpu.SemaphoreType.DMA,
        pltpu.SemaphoreType.DMA,
        pltpu.SemaphoreType.DMA,
        pltpu.SemaphoreType.DMA,
        pltpu.SemaphoreType.DMA,
    ],
)


def _kb_body(chunks, ytab, srcp, dstp, out, acc_sp, sidx, didx, sidxb, didxb,
             rows0, rows1, gsem0, gsem1, ssem0, ssem1, isem):
  c = lax.axis_index("c")
  sid = lax.axis_index("s")
  erow0 = sid * BPT
  r0 = sid * RPT
  rows = (rows0, rows1)
  idxp = ((sidx, didx), (sidxb, didxb))
  gsems = (gsem0, gsem1)
  ssems = (ssem0, ssem1)
  for pos, ci in enumerate(chunks):
    assigned = (pos * N_CORES) // len(chunks)

    @pl.when(c == assigned)
    def _(ci=ci, pos=pos):
      _agg_chunk(ytab.at[ci], srcp, dstp, out.at[pos, pl.ds(r0, RPT), :],
                 acc_sp, idxp, rows, gsems, ssems, isem, erow0, r0)


def _make_kb(chunks):
  return pl.kernel(
      functools.partial(_kb_body, chunks),
      out_type=jax.ShapeDtypeStruct((len(chunks), N_PAD, LANES), jnp.float32),
      mesh=_MESH,
      scratch_types=[
        pltpu.VMEM_SHARED((N_PAD, LANES), jnp.float32),
        pltpu.VMEM((IBLK, LANES), jnp.int32),
        pltpu.VMEM((IBLK, LANES), jnp.int32),
        pltpu.VMEM((IBLK, LANES), jnp.int32),
        pltpu.VMEM((IBLK, LANES), jnp.int32),
        pltpu.VMEM((LANES, LANES), jnp.float32),
        pltpu.VMEM((LANES, LANES), jnp.float32),
        pltpu.SemaphoreType.DMA,
        pltpu.SemaphoreType.DMA,
        pltpu.SemaphoreType.DMA,
        pltpu.SemaphoreType.DMA,
        pltpu.SemaphoreType.DMA,
    ],
  )


_kb_a = _make_kb((0, 2))
_kb_b = _make_kb((1, 3))


_ROWB = 2048


def _mm_body(g_ref, w_ref, dis_ref, s_ref, b_ref, o_ref, acc_ref,
             *, nk, relu, chunked):
  kc = pl.program_id(1)

  @pl.when(kc == 0)
  def _():
    acc_ref[...] = jnp.zeros_like(acc_ref)

  acc_ref[...] += jnp.dot(g_ref[0].astype(jnp.bfloat16),
                          w_ref[...].astype(jnp.bfloat16),
                          preferred_element_type=jnp.float32)

  @pl.when(kc == nk - 1)
  def _():
    t = dis_ref[...] * acc_ref[...] + s_ref[...] * b_ref[...]
    if relu:
      t = jnp.maximum(t, 0.0)
      t = dis_ref[...] * t
    if chunked:
      for co in range(D_HID // LANES):
        o_ref[co] = t[:, co * LANES:(co + 1) * LANES]
    else:
      o_ref[...] = t


def _k3(g, W, dis2d, s2d, b2d, relu, chunked):
  nk = g.shape[0]
  nco = D_HID // LANES
  if chunked:
    rowb = _ROWB
    nrb = N_PAD // rowb
    out_shape = jax.ShapeDtypeStruct((nco, N_PAD, LANES), jnp.float32)
    out_spec = pl.BlockSpec((nco, rowb, LANES), lambda rb, kc: (0, rb, 0))
  else:
    rowb = 2000
    nrb = N // rowb
    out_shape = jax.ShapeDtypeStruct((N, D_HID), jnp.float32)
    out_spec = pl.BlockSpec((rowb, D_HID), lambda rb, kc: (rb, 0))
  return pl.pallas_call(
      functools.partial(_mm_body, nk=nk, relu=relu, chunked=chunked),
      grid=(nrb, nk),
      in_specs=[
          pl.BlockSpec((1, rowb, LANES), lambda rb, kc: (kc, rb, 0)),
          pl.BlockSpec((LANES, D_HID), lambda rb, kc: (kc, 0)),
          pl.BlockSpec((rowb, 1), lambda rb, kc: (rb, 0)),
          pl.BlockSpec((rowb, 1), lambda rb, kc: (rb, 0)),
          pl.BlockSpec((1, D_HID), lambda rb, kc: (0, 0)),
      ],
      out_specs=out_spec,
      out_shape=out_shape,
      scratch_shapes=[pltpu.VMEM((rowb, D_HID), jnp.float32)],
      compiler_params=pltpu.CompilerParams(
          dimension_semantics=("parallel", "arbitrary")),
  )(g, W, dis2d, s2d, b2d)


_ZROWB = 2000


def _mmp_body(g_ref, w_ref, o_ref, acc_ref, *, nk):
  kc = pl.program_id(1)

  @pl.when(kc == 0)
  def _():
    acc_ref[...] = jnp.zeros_like(acc_ref)

  acc_ref[...] += jnp.dot(g_ref[0].astype(jnp.bfloat16),
                          w_ref[0].astype(jnp.bfloat16),
                          preferred_element_type=jnp.float32)

  @pl.when(kc == nk - 1)
  def _():
    o_ref[...] = acc_ref[...]


def _mmf_body(g_ref, w_ref, p_ref, dis_ref, s_ref, b_ref, o_ref, acc_ref,
              *, nk):
  kc = pl.program_id(1)

  @pl.when(kc == 0)
  def _():
    acc_ref[...] = p_ref[...]

  acc_ref[...] += jnp.dot(g_ref[0].astype(jnp.bfloat16),
                          w_ref[0].astype(jnp.bfloat16),
                          preferred_element_type=jnp.float32)

  @pl.when(kc == nk - 1)
  def _():
    o_ref[...] = dis_ref[...] * acc_ref[...] + s_ref[...] * b_ref[...]


def _k3z_partial(g, W3):
  nk = g.shape[0]
  return pl.pallas_call(
      functools.partial(_mmp_body, nk=nk),
      grid=(N // _ZROWB, nk),
      in_specs=[
          pl.BlockSpec((1, _ZROWB, LANES), lambda rb, kc: (kc, rb, 0)),
          pl.BlockSpec((1, LANES, D_HID), lambda rb, kc: (kc, 0, 0)),
      ],
      out_specs=pl.BlockSpec((_ZROWB, D_HID), lambda rb, kc: (rb, 0)),
      out_shape=jax.ShapeDtypeStruct((N, D_HID), jnp.float32),
      scratch_shapes=[pltpu.VMEM((_ZROWB, D_HID), jnp.float32)],
      compiler_params=pltpu.CompilerParams(
          dimension_semantics=("parallel", "arbitrary")),
  )(g, W3)


def _k3z_final(g, W3, p, dis2d, s2d, b2d):
  nk = g.shape[0]
  return pl.pallas_call(
      functools.partial(_mmf_body, nk=nk),
      grid=(N // _ZROWB, nk),
      in_specs=[
          pl.BlockSpec((1, _ZROWB, LANES), lambda rb, kc: (kc, rb, 0)),
          pl.BlockSpec((1, LANES, D_HID), lambda rb, kc: (kc, 0, 0)),
          pl.BlockSpec((_ZROWB, D_HID), lambda rb, kc: (rb, 0)),
          pl.BlockSpec((_ZROWB, 1), lambda rb, kc: (rb, 0)),
          pl.BlockSpec((_ZROWB, 1), lambda rb, kc: (rb, 0)),
          pl.BlockSpec((1, D_HID), lambda rb, kc: (0, 0)),
      ],
      out_specs=pl.BlockSpec((_ZROWB, D_HID), lambda rb, kc: (rb, 0)),
      out_shape=jax.ShapeDtypeStruct((N, D_HID), jnp.float32),
      scratch_shapes=[pltpu.VMEM((_ZROWB, D_HID), jnp.float32)],
      compiler_params=pltpu.CompilerParams(
          dimension_semantics=("parallel", "arbitrary")),
  )(g, W3, p, dis2d, s2d, b2d)


def kernel(x, edge_index, W1, b1, W2, b2):
  src = edge_index[0].astype(jnp.int32)
  dst = edge_index[1].astype(jnp.int32)
  pad = (jnp.arange(E_PAD - E, dtype=jnp.int32) % N_PAD_ROWS) + N
  srcp = jnp.concatenate([src, pad]).reshape(E_BATCHES, LANES)
  dstp = jnp.concatenate([dst, pad]).reshape(E_BATCHES, LANES)

  xpad = jnp.pad(x, ((0, N_PAD - N), (0, 0)))
  dis, s, _, g1 = _ka(xpad, srcp, dstp)
  dis2d = dis.reshape(N_PAD, 1)
  s2d = s.reshape(N_PAD, 1)

  y2 = _k3(g1, W1, dis2d, s2d, b1.reshape(1, D_HID),
           relu=True, chunked=True)

  W2r = W2.reshape(D_HID // LANES, LANES, D_HID)
  g2a = _kb_a(y2, srcp, dstp)
  g2b = _kb_b(y2, srcp, dstp)
  pa = _k3z_partial(g2a, jnp.stack((W2r[0], W2r[2])))
  z = _k3z_final(g2b, jnp.stack((W2r[1], W2r[3])), pa,
                 dis2d, s2d, b2.reshape(1, D_HID))
  return z


# speedup vs baseline: 1.1389x; 1.1389x over previous
import functools

import jax
import jax.numpy as jnp
from jax import lax
from jax.experimental import pallas as pl
from jax.experimental.pallas import tpu as pltpu
from jax.experimental.pallas import tpu_sc as plsc

N = 10000
E = 160000
D_IN = 256
D_HID = 512

N_PAD = 10240
N_PAD_ROWS = N_PAD - N
LANES = 128
N_SUBCORES = 16
N_CORES = 2
BPT = -(-E // (N_SUBCORES * LANES * 8)) * 8
E_BATCHES = BPT * N_SUBCORES
E_PAD = E_BATCHES * LANES
RPT = N_PAD // N_SUBCORES
IBLK = 16
N_IBLK = BPT // IBLK
XSUB = RPT // LANES

_MESH = plsc.VectorSubcoreMesh(
    core_axis_name="c", subcore_axis_name="s",
    num_cores=N_CORES, num_subcores=N_SUBCORES)


def _rsqrt16(x):
  i = lax.bitcast_convert_type(x, jnp.int32)
  i = 0x5F3759DF - lax.shift_right_arithmetic(i, 1)
  y = lax.bitcast_convert_type(i, jnp.float32)
  for _ in range(3):
    y = y * (1.5 - 0.5 * x * y * y)
  return y


def _agg_chunk(tab, srcp, dstp, out_slice, acc_sp, idxp, rows, gsems, ssems,
               isem, erow0, r0):
  pltpu.sync_copy(tab.at[pl.ds(r0, RPT), :], acc_sp.at[pl.ds(r0, RPT), :])
  plsc.subcore_barrier()

  _edge_pipeline(tab, acc_sp, srcp, dstp, idxp, rows, gsems, ssems, isem,
                 erow0)

  plsc.subcore_barrier()
  pltpu.sync_copy(acc_sp.at[pl.ds(r0, RPT), :], out_slice)
  plsc.subcore_barrier()


def _edge_pipeline(tab, accum, srcp, dstp, idxp, bufs, gsems, ssems, isem,
                   erow0):
  (sidx0, didx0), (sidx1, didx1) = idxp
  sidxs = (sidx0, sidx1)
  didxs = (didx0, didx1)
  pltpu.sync_copy(srcp.at[pl.ds(erow0, IBLK), :], sidx0)
  pltpu.sync_copy(dstp.at[pl.ds(erow0, IBLK), :], didx0)

  gd = [None, None]
  sd = [None, None]
  id_ = [None, None]
  gd[0] = pltpu.async_copy(tab.at[sidx0.at[0]], bufs[0], gsems[0])
  for blk in range(N_IBLK):
    cur = blk % 2
    if blk >= 1:
      sd[(blk * IBLK - 1) % 2].wait()
    if blk + 1 < N_IBLK:
      nxt = (blk + 1) % 2
      off = erow0 + (blk + 1) * IBLK
      id_[0] = pltpu.async_copy(srcp.at[pl.ds(off, IBLK), :], sidxs[nxt], isem)
      id_[1] = pltpu.async_copy(dstp.at[pl.ds(off, IBLK), :], didxs[nxt], isem)
    for b in range(IBLK):
      k = blk * IBLK + b
      if k >= 1 and b >= 1:
        sd[(k - 1) % 2].wait()
      if k + 1 < BPT:
        if b + 1 < IBLK:
          nsidx, nb = sidxs[cur], b + 1
        else:
          id_[0].wait()
          id_[1].wait()
          nsidx, nb = sidxs[(blk + 1) % 2], 0
        gd[(k + 1) % 2] = pltpu.async_copy(
            tab.at[nsidx.at[nb]], bufs[(k + 1) % 2], gsems[(k + 1) % 2])
      gd[k % 2].wait()
      sd[k % 2] = pltpu.async_copy(
          bufs[k % 2], accum.at[didxs[cur].at[b]], ssems[k % 2], add=True)
  sd[(BPT - 1) % 2].wait()


def _ka_body(xpad, srcp, dstp, dis_hbm, s_hbm, y1_hbm, g1_hbm,
             acc_sp, deg_sp, t_sp, dis_sp,
             sidx, didx, sidxb, didxb, rows0, rows1, ones_v, vals_v, vals_b,
             buf_a, buf_b, gsem0, gsem1, ssem0, ssem1, isem):
  c = lax.axis_index("c")
  sid = lax.axis_index("s")
  erow0 = sid * BPT
  r0 = sid * RPT
  rows = (rows0, rows1)
  idxp = ((sidx, didx), (sidxb, didxb))
  gsems = (gsem0, gsem1)
  ssems = (ssem0, ssem1)

  @pl.loop(0, RPT // 16)
  def _zero(i):
    buf_a[pl.ds(i * 16, 16)] = jnp.zeros((16,), jnp.float32)

  pltpu.sync_copy(buf_a, deg_sp.at[pl.ds(r0, RPT)])
  pltpu.sync_copy(buf_a, t_sp.at[pl.ds(r0, RPT)])
  for i in range(LANES // 16):
    ones_v[pl.ds(i * 16, 16)] = jnp.ones((16,), jnp.float32)
  plsc.subcore_barrier()

  didxs = (didx, didxb)
  pltpu.sync_copy(dstp.at[pl.ds(erow0, IBLK), :], didxs[0])
  for blk in range(N_IBLK):
    cur = blk % 2
    idxd = None
    if blk + 1 < N_IBLK:
      idxd = pltpu.async_copy(
          dstp.at[pl.ds(erow0 + (blk + 1) * IBLK, IBLK), :],
          didxs[(blk + 1) % 2], isem)
    descs = [pltpu.async_copy(ones_v, deg_sp.at[didxs[cur].at[b]], ssem0,
                              add=True)
             for b in range(IBLK)]
    for d in descs:
      d.wait()
    if idxd is not None:
      idxd.wait()
  plsc.subcore_barrier()

  pltpu.sync_copy(deg_sp.at[pl.ds(r0, RPT)], buf_a)

  @pl.loop(0, RPT // 16)
  def _dis(i):
    d = buf_a[pl.ds(i * 16, 16)] + 1.0
    buf_b[pl.ds(i * 16, 16)] = _rsqrt16(d)

  pltpu.sync_copy(buf_b, dis_sp.at[pl.ds(r0, RPT)])

  @pl.when(c == 0)
  def _():
    pltpu.sync_copy(buf_b, dis_hbm.at[pl.ds(r0, RPT)])

  plsc.subcore_barrier()

  _edge_pipeline(dis_sp, t_sp, srcp, dstp, idxp, (vals_v, vals_b),
                 gsems, ssems, isem, erow0)
  plsc.subcore_barrier()

  pltpu.sync_copy(t_sp.at[pl.ds(r0, RPT)], buf_a)

  @pl.loop(0, RPT // 16)
  def _s(i):
    d = buf_b[pl.ds(i * 16, 16)]
    buf_a[pl.ds(i * 16, 16)] = d * (buf_a[pl.ds(i * 16, 16)] + d)

  @pl.when(c == 0)
  def _():
    pltpu.sync_copy(buf_a, s_hbm.at[pl.ds(r0, RPT)])

  for sub in range(XSUB):
    rbase = r0 + sub * LANES
    pltpu.sync_copy(
        xpad.at[pl.ds(rbase, LANES), pl.ds(c * LANES, LANES)], rows0)

    @pl.loop(0, LANES // 16)
    def _scale(rg):
      d16 = buf_b[pl.ds(sub * LANES + rg * 16, 16)]
      for l in range(16):
        d = d16[l]
        rr = rg * 16 + l
        for j in range(LANES // 16):
          rows0[rr, pl.ds(j * 16, 16)] = rows0[rr, pl.ds(j * 16, 16)] * d

    pltpu.sync_copy(rows0, y1_hbm.at[c, pl.ds(rbase, LANES), :])
  plsc.subcore_barrier()

  _agg_chunk(y1_hbm.at[c], srcp, dstp, g1_hbm.at[c, pl.ds(r0, RPT), :],
             acc_sp, idxp, rows, gsems, ssems, isem, erow0, r0)


_ka = pl.kernel(
    _ka_body,
    out_type=[jax.ShapeDtypeStruct((N_PAD,), jnp.float32),
              jax.ShapeDtypeStruct((N_PAD,), jnp.float32),
              jax.ShapeDtypeStruct((D_IN // LANES, N_PAD, LANES), jnp.float32),
              jax.ShapeDtypeStruct((D_IN // LANES, N_PAD, LANES), jnp.float32)],
    mesh=_MESH,
    scratch_types=[
        pltpu.VMEM_SHARED((N_PAD, LANES), jnp.float32),
        pltpu.VMEM_SHARED((N_PAD,), jnp.float32),
        pltpu.VMEM_SHARED((N_PAD,), jnp.float32),
        pltpu.VMEM_SHARED((N_PAD,), jnp.float32),
        pltpu.VMEM((IBLK, LANES), jnp.int32),
        pltpu.VMEM((IBLK, LANES), jnp.int32),
        pltpu.VMEM((IBLK, LANES), jnp.int32),
        pltpu.VMEM((IBLK, LANES), jnp.int32),
        pltpu.VMEM((LANES, LANES), jnp.float32),
        pltpu.VMEM((LANES, LANES), jnp.float32),
        pltpu.VMEM((LANES,), jnp.float32),
        pltpu.VMEM((LANES,), jnp.float32),
        pltpu.VMEM((LANES,), jnp.float32),
        pltpu.VMEM((RPT,), jnp.float32),
        pltpu.VMEM((RPT,), jnp.float32),
        pltpu.SemaphoreType.DMA,
        pltpu.SemaphoreType.DMA,
        pltpu.SemaphoreType.DMA,
        pltpu.SemaphoreType.DMA,
        pltpu.SemaphoreType.DMA,
    ],
)


def _kb_body(chunks, ytab, srcp, dstp, out, acc_sp, sidx, didx, sidxb, didxb,
             rows0, rows1, gsem0, gsem1, ssem0, ssem1, isem):
  c = lax.axis_index("c")
  sid = lax.axis_index("s")
  erow0 = sid * BPT
  r0 = sid * RPT
  rows = (rows0, rows1)
  idxp = ((sidx, didx), (sidxb, didxb))
  gsems = (gsem0, gsem1)
  ssems = (ssem0, ssem1)
  for pos, ci in enumerate(chunks):
    assigned = (pos * N_CORES) // len(chunks)

    @pl.when(c == assigned)
    def _(ci=ci, pos=pos):
      _agg_chunk(ytab.at[ci], srcp, dstp, out.at[pos, pl.ds(r0, RPT), :],
                 acc_sp, idxp, rows, gsems, ssems, isem, erow0, r0)


def _make_kb(chunks):
  return pl.kernel(
      functools.partial(_kb_body, chunks),
      out_type=jax.ShapeDtypeStruct((len(chunks), N_PAD, LANES), jnp.float32),
      mesh=_MESH,
      scratch_types=[
        pltpu.VMEM_SHARED((N_PAD, LANES), jnp.float32),
        pltpu.VMEM((IBLK, LANES), jnp.int32),
        pltpu.VMEM((IBLK, LANES), jnp.int32),
        pltpu.VMEM((IBLK, LANES), jnp.int32),
        pltpu.VMEM((IBLK, LANES), jnp.int32),
        pltpu.VMEM((LANES, LANES), jnp.float32),
        pltpu.VMEM((LANES, LANES), jnp.float32),
        pltpu.SemaphoreType.DMA,
        pltpu.SemaphoreType.DMA,
        pltpu.SemaphoreType.DMA,
        pltpu.SemaphoreType.DMA,
        pltpu.SemaphoreType.DMA,
    ],
  )


_kb = _make_kb((0, 1, 2, 3))


_ROWB = 2048


def _mm_body(g_ref, w_ref, dis_ref, s_ref, b_ref, o_ref, acc_ref,
             *, nk, relu, chunked):
  kc = pl.program_id(1)

  @pl.when(kc == 0)
  def _():
    acc_ref[...] = jnp.zeros_like(acc_ref)

  acc_ref[...] += jnp.dot(g_ref[0].astype(jnp.bfloat16),
                          w_ref[...].astype(jnp.bfloat16),
                          preferred_element_type=jnp.float32)

  @pl.when(kc == nk - 1)
  def _():
    t = dis_ref[...] * acc_ref[...] + s_ref[...] * b_ref[...]
    if relu:
      t = jnp.maximum(t, 0.0)
      t = dis_ref[...] * t
    if chunked:
      for co in range(D_HID // LANES):
        o_ref[co] = t[:, co * LANES:(co + 1) * LANES]
    else:
      o_ref[...] = t


def _k3(g, W, dis2d, s2d, b2d, relu, chunked):
  nk = g.shape[0]
  nco = D_HID // LANES
  if chunked:
    rowb = _ROWB
    nrb = N_PAD // rowb
    out_shape = jax.ShapeDtypeStruct((nco, N_PAD, LANES), jnp.float32)
    out_spec = pl.BlockSpec((nco, rowb, LANES), lambda rb, kc: (0, rb, 0))
  else:
    rowb = 2000
    nrb = N // rowb
    out_shape = jax.ShapeDtypeStruct((N, D_HID), jnp.float32)
    out_spec = pl.BlockSpec((rowb, D_HID), lambda rb, kc: (rb, 0))
  return pl.pallas_call(
      functools.partial(_mm_body, nk=nk, relu=relu, chunked=chunked),
      grid=(nrb, nk),
      in_specs=[
          pl.BlockSpec((1, rowb, LANES), lambda rb, kc: (kc, rb, 0)),
          pl.BlockSpec((LANES, D_HID), lambda rb, kc: (kc, 0)),
          pl.BlockSpec((rowb, 1), lambda rb, kc: (rb, 0)),
          pl.BlockSpec((rowb, 1), lambda rb, kc: (rb, 0)),
          pl.BlockSpec((1, D_HID), lambda rb, kc: (0, 0)),
      ],
      out_specs=out_spec,
      out_shape=out_shape,
      scratch_shapes=[pltpu.VMEM((rowb, D_HID), jnp.float32)],
      compiler_params=pltpu.CompilerParams(
          dimension_semantics=("parallel", "arbitrary")),
  )(g, W, dis2d, s2d, b2d)


_ZROWB = 2000


def _mmp_body(g_ref, w_ref, o_ref, acc_ref, *, nk):
  kc = pl.program_id(1)

  @pl.when(kc == 0)
  def _():
    acc_ref[...] = jnp.zeros_like(acc_ref)

  acc_ref[...] += jnp.dot(g_ref[0].astype(jnp.bfloat16),
                          w_ref[0].astype(jnp.bfloat16),
                          preferred_element_type=jnp.float32)

  @pl.when(kc == nk - 1)
  def _():
    o_ref[...] = acc_ref[...]


def _mmf_body(g_ref, w_ref, p_ref, dis_ref, s_ref, b_ref, o_ref, acc_ref,
              *, nk):
  kc = pl.program_id(1)

  @pl.when(kc == 0)
  def _():
    acc_ref[...] = p_ref[...]

  acc_ref[...] += jnp.dot(g_ref[0].astype(jnp.bfloat16),
                          w_ref[0].astype(jnp.bfloat16),
                          preferred_element_type=jnp.float32)

  @pl.when(kc == nk - 1)
  def _():
    o_ref[...] = dis_ref[...] * acc_ref[...] + s_ref[...] * b_ref[...]


def _k3z_partial(g, W3):
  nk = g.shape[0]
  return pl.pallas_call(
      functools.partial(_mmp_body, nk=nk),
      grid=(N // _ZROWB, nk),
      in_specs=[
          pl.BlockSpec((1, _ZROWB, LANES), lambda rb, kc: (kc, rb, 0)),
          pl.BlockSpec((1, LANES, D_HID), lambda rb, kc: (kc, 0, 0)),
      ],
      out_specs=pl.BlockSpec((_ZROWB, D_HID), lambda rb, kc: (rb, 0)),
      out_shape=jax.ShapeDtypeStruct((N, D_HID), jnp.float32),
      scratch_shapes=[pltpu.VMEM((_ZROWB, D_HID), jnp.float32)],
      compiler_params=pltpu.CompilerParams(
          dimension_semantics=("parallel", "arbitrary")),
  )(g, W3)


def _k3z_final(g, W3, p, dis2d, s2d, b2d):
  nk = g.shape[0]
  return pl.pallas_call(
      functools.partial(_mmf_body, nk=nk),
      grid=(N // _ZROWB, nk),
      in_specs=[
          pl.BlockSpec((1, _ZROWB, LANES), lambda rb, kc: (kc, rb, 0)),
          pl.BlockSpec((1, LANES, D_HID), lambda rb, kc: (kc, 0, 0)),
          pl.BlockSpec((_ZROWB, D_HID), lambda rb, kc: (rb, 0)),
          pl.BlockSpec((_ZROWB, 1), lambda rb, kc: (rb, 0)),
          pl.BlockSpec((_ZROWB, 1), lambda rb, kc: (rb, 0)),
          pl.BlockSpec((1, D_HID), lambda rb, kc: (0, 0)),
      ],
      out_specs=pl.BlockSpec((_ZROWB, D_HID), lambda rb, kc: (rb, 0)),
      out_shape=jax.ShapeDtypeStruct((N, D_HID), jnp.float32),
      scratch_shapes=[pltpu.VMEM((_ZROWB, D_HID), jnp.float32)],
      compiler_params=pltpu.CompilerParams(
          dimension_semantics=("parallel", "arbitrary")),
  )(g, W3, p, dis2d, s2d, b2d)


def kernel(x, edge_index, W1, b1, W2, b2):
  src = edge_index[0].astype(jnp.int32)
  dst = edge_index[1].astype(jnp.int32)
  pad = (jnp.arange(E_PAD - E, dtype=jnp.int32) % N_PAD_ROWS) + N
  srcp = jnp.concatenate([src, pad]).reshape(E_BATCHES, LANES)
  dstp = jnp.concatenate([dst, pad]).reshape(E_BATCHES, LANES)

  xpad = jnp.pad(x, ((0, N_PAD - N), (0, 0)))
  dis, s, _, g1 = _ka(xpad, srcp, dstp)
  dis2d = dis.reshape(N_PAD, 1)
  s2d = s.reshape(N_PAD, 1)

  y2 = _k3(g1, W1, dis2d, s2d, b1.reshape(1, D_HID),
           relu=True, chunked=True)

  g2 = _kb(y2, srcp, dstp)
  z = _k3(g2, W2, dis2d, s2d, b2.reshape(1, D_HID),
          relu=False, chunked=False)
  return z


# speedup vs baseline: 1.1473x; 1.0074x over previous
import functools

import jax
import jax.numpy as jnp
from jax import lax
from jax.experimental import pallas as pl
from jax.experimental.pallas import tpu as pltpu
from jax.experimental.pallas import tpu_sc as plsc

N = 10000
E = 160000
D_IN = 256
D_HID = 512

N_PAD = 10240
N_PAD_ROWS = N_PAD - N
LANES = 128
N_SUBCORES = 16
N_CORES = 2
BPT = -(-E // (N_SUBCORES * LANES * 8)) * 8
E_BATCHES = BPT * N_SUBCORES
E_PAD = E_BATCHES * LANES
RPT = N_PAD // N_SUBCORES
IBLK = 16
N_IBLK = BPT // IBLK
XSUB = RPT // LANES

_MESH = plsc.VectorSubcoreMesh(
    core_axis_name="c", subcore_axis_name="s",
    num_cores=N_CORES, num_subcores=N_SUBCORES)


def _rsqrt16(x):
  i = lax.bitcast_convert_type(x, jnp.int32)
  i = 0x5F3759DF - lax.shift_right_arithmetic(i, 1)
  y = lax.bitcast_convert_type(i, jnp.float32)
  for _ in range(3):
    y = y * (1.5 - 0.5 * x * y * y)
  return y


def _agg_chunk(tab, srcp, dstp, out_slice, acc_sp, idxp, rows, gsems, ssems,
               isem, erow0, r0, do_init=True):
  if do_init:
    pltpu.sync_copy(tab.at[pl.ds(r0, RPT), :], acc_sp.at[pl.ds(r0, RPT), :])
    plsc.subcore_barrier()

  _edge_pipeline(tab, acc_sp, srcp, dstp, idxp, rows, gsems, ssems, isem,
                 erow0)

  plsc.subcore_barrier()
  pltpu.sync_copy(acc_sp.at[pl.ds(r0, RPT), :], out_slice)
  plsc.subcore_barrier()


def _edge_pipeline(tab, accum, srcp, dstp, idxp, bufs, gsems, ssems, isem,
                   erow0):
  (sidx0, didx0), (sidx1, didx1) = idxp
  sidxs = (sidx0, sidx1)
  didxs = (didx0, didx1)
  pltpu.sync_copy(srcp.at[pl.ds(erow0, IBLK), :], sidx0)
  pltpu.sync_copy(dstp.at[pl.ds(erow0, IBLK), :], didx0)

  gd = [None, None]
  sd = [None, None]
  id_ = [None, None]
  gd[0] = pltpu.async_copy(tab.at[sidx0.at[0]], bufs[0], gsems[0])
  for blk in range(N_IBLK):
    cur = blk % 2
    if blk >= 1:
      sd[(blk * IBLK - 1) % 2].wait()
    if blk + 1 < N_IBLK:
      nxt = (blk + 1) % 2
      off = erow0 + (blk + 1) * IBLK
      id_[0] = pltpu.async_copy(srcp.at[pl.ds(off, IBLK), :], sidxs[nxt], isem)
      id_[1] = pltpu.async_copy(dstp.at[pl.ds(off, IBLK), :], didxs[nxt], isem)
    for b in range(IBLK):
      k = blk * IBLK + b
      if k >= 1 and b >= 1:
        sd[(k - 1) % 2].wait()
      if k + 1 < BPT:
        if b + 1 < IBLK:
          nsidx, nb = sidxs[cur], b + 1
        else:
          id_[0].wait()
          id_[1].wait()
          nsidx, nb = sidxs[(blk + 1) % 2], 0
        gd[(k + 1) % 2] = pltpu.async_copy(
            tab.at[nsidx.at[nb]], bufs[(k + 1) % 2], gsems[(k + 1) % 2])
      gd[k % 2].wait()
      sd[k % 2] = pltpu.async_copy(
          bufs[k % 2], accum.at[didxs[cur].at[b]], ssems[k % 2], add=True)
  sd[(BPT - 1) % 2].wait()


def _ka_body(xpad, srcp, dstp, dis_hbm, s_hbm, y1_hbm, g1_hbm,
             acc_sp, deg_sp, t_sp, dis_sp,
             sidx, didx, sidxb, didxb, rows0, rows1, ones_v, vals_v, vals_b,
             buf_a, buf_b, gsem0, gsem1, ssem0, ssem1, isem):
  c = lax.axis_index("c")
  sid = lax.axis_index("s")
  erow0 = sid * BPT
  r0 = sid * RPT
  rows = (rows0, rows1)
  idxp = ((sidx, didx), (sidxb, didxb))
  gsems = (gsem0, gsem1)
  ssems = (ssem0, ssem1)

  @pl.loop(0, RPT // 16)
  def _zero(i):
    buf_a[pl.ds(i * 16, 16)] = jnp.zeros((16,), jnp.float32)

  pltpu.sync_copy(buf_a, deg_sp.at[pl.ds(r0, RPT)])
  pltpu.sync_copy(buf_a, t_sp.at[pl.ds(r0, RPT)])
  for i in range(LANES // 16):
    ones_v[pl.ds(i * 16, 16)] = jnp.ones((16,), jnp.float32)
  plsc.subcore_barrier()

  didxs = (didx, didxb)
  pltpu.sync_copy(dstp.at[pl.ds(erow0, IBLK), :], didxs[0])
  for blk in range(N_IBLK):
    cur = blk % 2
    idxd = None
    if blk + 1 < N_IBLK:
      idxd = pltpu.async_copy(
          dstp.at[pl.ds(erow0 + (blk + 1) * IBLK, IBLK), :],
          didxs[(blk + 1) % 2], isem)
    descs = [pltpu.async_copy(ones_v, deg_sp.at[didxs[cur].at[b]], ssem0,
                              add=True)
             for b in range(IBLK)]
    for d in descs:
      d.wait()
    if idxd is not None:
      idxd.wait()
  plsc.subcore_barrier()

  pltpu.sync_copy(deg_sp.at[pl.ds(r0, RPT)], buf_a)

  @pl.loop(0, RPT // 16)
  def _dis(i):
    d = buf_a[pl.ds(i * 16, 16)] + 1.0
    buf_b[pl.ds(i * 16, 16)] = _rsqrt16(d)

  pltpu.sync_copy(buf_b, dis_sp.at[pl.ds(r0, RPT)])

  @pl.when(c == 0)
  def _():
    pltpu.sync_copy(buf_b, dis_hbm.at[pl.ds(r0, RPT)])

  plsc.subcore_barrier()

  _edge_pipeline(dis_sp, t_sp, srcp, dstp, idxp, (vals_v, vals_b),
                 gsems, ssems, isem, erow0)
  plsc.subcore_barrier()

  pltpu.sync_copy(t_sp.at[pl.ds(r0, RPT)], buf_a)

  @pl.loop(0, RPT // 16)
  def _s(i):
    d = buf_b[pl.ds(i * 16, 16)]
    buf_a[pl.ds(i * 16, 16)] = d * (buf_a[pl.ds(i * 16, 16)] + d)

  @pl.when(c == 0)
  def _():
    pltpu.sync_copy(buf_a, s_hbm.at[pl.ds(r0, RPT)])

  for sub in range(XSUB):
    rbase = r0 + sub * LANES
    pltpu.sync_copy(
        xpad.at[pl.ds(rbase, LANES), pl.ds(c * LANES, LANES)], rows0)

    @pl.loop(0, LANES // 16)
    def _scale(rg):
      d16 = buf_b[pl.ds(sub * LANES + rg * 16, 16)]
      for l in range(16):
        d = d16[l]
        rr = rg * 16 + l
        for j in range(LANES // 16):
          rows0[rr, pl.ds(j * 16, 16)] = rows0[rr, pl.ds(j * 16, 16)] * d

    pltpu.sync_copy(rows0, y1_hbm.at[c, pl.ds(rbase, LANES), :])
    pltpu.sync_copy(rows0, acc_sp.at[pl.ds(rbase, LANES), :])
  plsc.subcore_barrier()

  _agg_chunk(y1_hbm.at[c], srcp, dstp, g1_hbm.at[c, pl.ds(r0, RPT), :],
             acc_sp, idxp, rows, gsems, ssems, isem, erow0, r0,
             do_init=False)


_ka = pl.kernel(
    _ka_body,
    out_type=[jax.ShapeDtypeStruct((N_PAD,), jnp.float32),
              jax.ShapeDtypeStruct((N_PAD,), jnp.float32),
              jax.ShapeDtypeStruct((D_IN // LANES, N_PAD, LANES), jnp.float32),
              jax.ShapeDtypeStruct((D_IN // LANES, N_PAD, LANES), jnp.float32)],
    mesh=_MESH,
    scratch_types=[
        pltpu.VMEM_SHARED((N_PAD, LANES), jnp.float32),
        pltpu.VMEM_SHARED((N_PAD,), jnp.float32),
        pltpu.VMEM_SHARED((N_PAD,), jnp.float32),
        pltpu.VMEM_SHARED((N_PAD,), jnp.float32),
        pltpu.VMEM((IBLK, LANES), jnp.int32),
        pltpu.VMEM((IBLK, LANES), jnp.int32),
        pltpu.VMEM((IBLK, LANES), jnp.int32),
        pltpu.VMEM((IBLK, LANES), jnp.int32),
        pltpu.VMEM((LANES, LANES), jnp.float32),
        pltpu.VMEM((LANES, LANES), jnp.float32),
        pltpu.VMEM((LANES,), jnp.float32),
        pltpu.VMEM((LANES,), jnp.float32),
        pltpu.VMEM((LANES,), jnp.float32),
        pltpu.VMEM((RPT,), jnp.float32),
        pltpu.VMEM((RPT,), jnp.float32),
        pltpu.SemaphoreType.DMA,
        pltpu.SemaphoreType.DMA,
        pltpu.SemaphoreType.DMA,
        pltpu.SemaphoreType.DMA,
        pltpu.SemaphoreType.DMA,
    ],
)


def _kb_body(chunks, ytab, srcp, dstp, out, acc_sp, sidx, didx, sidxb, didxb,
             rows0, rows1, gsem0, gsem1, ssem0, ssem1, isem):
  c = lax.axis_index("c")
  sid = lax.axis_index("s")
  erow0 = sid * BPT
  r0 = sid * RPT
  rows = (rows0, rows1)
  idxp = ((sidx, didx), (sidxb, didxb))
  gsems = (gsem0, gsem1)
  ssems = (ssem0, ssem1)
  for pos, ci in enumerate(chunks):
    assigned = (pos * N_CORES) // len(chunks)

    @pl.when(c == assigned)
    def _(ci=ci, pos=pos):
      _agg_chunk(ytab.at[ci], srcp, dstp, out.at[pos, pl.ds(r0, RPT), :],
                 acc_sp, idxp, rows, gsems, ssems, isem, erow0, r0)


def _make_kb(chunks):
  return pl.kernel(
      functools.partial(_kb_body, chunks),
      out_type=jax.ShapeDtypeStruct((len(chunks), N_PAD, LANES), jnp.float32),
      mesh=_MESH,
      scratch_types=[
        pltpu.VMEM_SHARED((N_PAD, LANES), jnp.float32),
        pltpu.VMEM((IBLK, LANES), jnp.int32),
        pltpu.VMEM((IBLK, LANES), jnp.int32),
        pltpu.VMEM((IBLK, LANES), jnp.int32),
        pltpu.VMEM((IBLK, LANES), jnp.int32),
        pltpu.VMEM((LANES, LANES), jnp.float32),
        pltpu.VMEM((LANES, LANES), jnp.float32),
        pltpu.SemaphoreType.DMA,
        pltpu.SemaphoreType.DMA,
        pltpu.SemaphoreType.DMA,
        pltpu.SemaphoreType.DMA,
        pltpu.SemaphoreType.DMA,
    ],
  )


_kb = _make_kb((0, 1, 2, 3))


_ROWB = 2048


def _mm_body(g_ref, w_ref, dis_ref, s_ref, b_ref, o_ref, acc_ref,
             *, nk, relu, chunked):
  kc = pl.program_id(1)

  @pl.when(kc == 0)
  def _():
    acc_ref[...] = jnp.zeros_like(acc_ref)

  acc_ref[...] += jnp.dot(g_ref[0].astype(jnp.bfloat16),
                          w_ref[...].astype(jnp.bfloat16),
                          preferred_element_type=jnp.float32)

  @pl.when(kc == nk - 1)
  def _():
    t = dis_ref[...] * acc_ref[...] + s_ref[...] * b_ref[...]
    if relu:
      t = jnp.maximum(t, 0.0)
      t = dis_ref[...] * t
    if chunked:
      for co in range(D_HID // LANES):
        o_ref[co] = t[:, co * LANES:(co + 1) * LANES]
    else:
      o_ref[...] = t


def _k3(g, W, dis2d, s2d, b2d, relu, chunked):
  nk = g.shape[0]
  nco = D_HID // LANES
  if chunked:
    rowb = _ROWB
    nrb = N_PAD // rowb
    out_shape = jax.ShapeDtypeStruct((nco, N_PAD, LANES), jnp.float32)
    out_spec = pl.BlockSpec((nco, rowb, LANES), lambda rb, kc: (0, rb, 0))
  else:
    rowb = 2000
    nrb = N // rowb
    out_shape = jax.ShapeDtypeStruct((N, D_HID), jnp.float32)
    out_spec = pl.BlockSpec((rowb, D_HID), lambda rb, kc: (rb, 0))
  return pl.pallas_call(
      functools.partial(_mm_body, nk=nk, relu=relu, chunked=chunked),
      grid=(nrb, nk),
      in_specs=[
          pl.BlockSpec((1, rowb, LANES), lambda rb, kc: (kc, rb, 0)),
          pl.BlockSpec((LANES, D_HID), lambda rb, kc: (kc, 0)),
          pl.BlockSpec((rowb, 1), lambda rb, kc: (rb, 0)),
          pl.BlockSpec((rowb, 1), lambda rb, kc: (rb, 0)),
          pl.BlockSpec((1, D_HID), lambda rb, kc: (0, 0)),
      ],
      out_specs=out_spec,
      out_shape=out_shape,
      scratch_shapes=[pltpu.VMEM((rowb, D_HID), jnp.float32)],
      compiler_params=pltpu.CompilerParams(
          dimension_semantics=("parallel", "arbitrary")),
  )(g, W, dis2d, s2d, b2d)


def kernel(x, edge_index, W1, b1, W2, b2):
  src = edge_index[0].astype(jnp.int32)
  dst = edge_index[1].astype(jnp.int32)
  pad = (jnp.arange(E_PAD - E, dtype=jnp.int32) % N_PAD_ROWS) + N
  srcp = jnp.concatenate([src, pad]).reshape(E_BATCHES, LANES)
  dstp = jnp.concatenate([dst, pad]).reshape(E_BATCHES, LANES)

  xpad = jnp.pad(x, ((0, N_PAD - N), (0, 0)))
  dis, s, _, g1 = _ka(xpad, srcp, dstp)
  dis2d = dis.reshape(N_PAD, 1)
  s2d = s.reshape(N_PAD, 1)

  y2 = _k3(g1, W1, dis2d, s2d, b1.reshape(1, D_HID),
           relu=True, chunked=True)

  g2 = _kb(y2, srcp, dstp)
  z = _k3(g2, W2, dis2d, s2d, b2.reshape(1, D_HID),
          relu=False, chunked=False)
  return z
